# baseline (speedup 1.0000x reference)
import functools

import jax
import jax.numpy as jnp
from jax import lax
from jax.experimental import pallas as pl
from jax.experimental.pallas import tpu as pltpu
from jax.experimental.pallas import tpu_sc as plsc

BATCH = 16384
EMB = 8
NUM = 1000000
FSTRIDE = 1 << 20
NC = 2
NS = 16
NW = NC * NS
BPW = BATCH // NW
CHUNK = 128
NCHUNK = BPW // CHUNK
IDX_ROWS = BATCH // CHUNK

UK = 16
UB = FSTRIDE // UK


def _untile_body(*refs):
    nt = len(refs) // 9
    ins = refs[:nt]
    outs = refs[nt:]
    for t in range(nt):
        for f in range(EMB):
            outs[t * EMB + f][...] = ins[t][f, :]


def _tc_untile(*tables):
    nt = len(tables)
    in_spec = pl.BlockSpec((EMB, UB), lambda k: (0, k))
    out_spec = pl.BlockSpec((UB,), lambda k: (k,))
    return pl.pallas_call(
        _untile_body,
        grid=(UK,),
        in_specs=[in_spec] * nt,
        out_specs=[out_spec] * (EMB * nt),
        out_shape=[jax.ShapeDtypeStruct((FSTRIDE,), jnp.float32)] * (EMB * nt),
    )(*tables)


def _sc_gather(idx2d, feats):
    mesh = plsc.VectorSubcoreMesh(core_axis_name="c", subcore_axis_name="s")
    out_t = jax.ShapeDtypeStruct((EMB, BATCH), jnp.float32)

    @functools.partial(
        pl.kernel,
        mesh=mesh,
        out_type=out_t,
        compiler_params=pltpu.CompilerParams(use_tc_tiling_on_sc=False),
        scratch_types=[
            pltpu.VMEM((NCHUNK, CHUNK), jnp.int32),
            pltpu.VMEM((EMB, BPW), jnp.float32),
            pltpu.SemaphoreType.DMA,
        ],
    )
    def k(i_hbm, *rest):
        s = rest[:EMB]
        o = rest[EMB]
        idx, rbuf, sem = rest[EMB + 1:]
        wid = lax.axis_index("s") * NC + lax.axis_index("c")
        base = wid * BPW
        row0 = wid * NCHUNK
        pltpu.sync_copy(i_hbm.at[pl.ds(row0, NCHUNK)], idx)
        copies = []
        for f in range(EMB):
            for g in range(NCHUNK):
                copies.append(pltpu.async_copy(
                    s[f].at[idx.at[g]],
                    rbuf.at[f, pl.ds(g * CHUNK, CHUNK)], sem))
        for c in copies:
            c.wait()
        pltpu.sync_copy(rbuf, o.at[:, pl.ds(base, BPW)])

    return k(idx2d, *feats)


BT = 2048


def _tc_body(um, im, uf, itf, w1u, w1i, b1c, w2, b2c, wah, waf, bac, out):
    h = jnp.maximum(
        jnp.dot(w1u[...], um[...], preferred_element_type=jnp.float32)
        + jnp.dot(w1i[...], im[...], preferred_element_type=jnp.float32)
        + b1c[...], 0.0)
    h2 = jnp.maximum(
        jnp.dot(w2[...], h, preferred_element_type=jnp.float32) + b2c[...], 0.0)
    mf = uf[...] * itf[...]
    logits = (jnp.dot(wah[...], h2, preferred_element_type=jnp.float32)
              + jnp.dot(waf[...], mf, preferred_element_type=jnp.float32)
              + bac[...])
    out[...] = jax.nn.sigmoid(logits)


def _tc_dense(u_mlp, i_mlp, u_mf, i_mf, w1u, w1i, b1c, w2, b2c, wah, waf, bac):
    grid = BATCH // BT
    emb_spec = pl.BlockSpec((EMB, BT), lambda i: (0, i))

    def wspec(shape):
        return pl.BlockSpec(shape, lambda i: (0, 0))

    return pl.pallas_call(
        _tc_body,
        grid=(grid,),
        in_specs=[
            emb_spec, emb_spec, emb_spec, emb_spec,
            wspec((32, EMB)), wspec((32, EMB)), wspec((32, 1)),
            wspec((EMB, 32)), wspec((EMB, 1)),
            wspec((1, EMB)), wspec((1, EMB)), wspec((1, 1)),
        ],
        out_specs=pl.BlockSpec((1, BT), lambda i: (0, i)),
        out_shape=jax.ShapeDtypeStruct((1, BATCH), jnp.float32),
    )(u_mlp, i_mlp, u_mf, i_mf, w1u, w1i, b1c, w2, b2c, wah, waf, bac)


def kernel(user_input, item_input, emb_user_mlp, emb_item_mlp,
           emb_user_mf, emb_item_mf, W1, b1, W2, b2, Wa, ba):
    u2d = user_input.astype(jnp.int32).reshape(IDX_ROWS, CHUNK)
    i2d = item_input.astype(jnp.int32).reshape(IDX_ROWS, CHUNK)
    f_um = _tc_untile(emb_user_mlp.T)
    gum = _sc_gather(u2d, f_um)
    f_uf = _tc_untile(emb_user_mf.T)
    guf = _sc_gather(u2d, f_uf)
    f_im = _tc_untile(emb_item_mlp.T)
    gim = _sc_gather(i2d, f_im)
    f_if = _tc_untile(emb_item_mf.T)
    gif = _sc_gather(i2d, f_if)
    out_t = _tc_dense(
        gum, gim, guf, gif,
        W1[:EMB].T, W1[EMB:].T, b1.reshape(32, 1),
        W2.T, b2.reshape(EMB, 1),
        Wa[:EMB].T, Wa[EMB:].T, ba.reshape(1, 1))
    return out_t.reshape(BATCH, 1)


# speedup vs baseline: 1.1026x; 1.1026x over previous
import functools

import jax
import jax.numpy as jnp
from jax import lax
from jax.experimental import pallas as pl
from jax.experimental.pallas import tpu as pltpu
from jax.experimental.pallas import tpu_sc as plsc

BATCH = 16384
EMB = 8
NUM = 1000000
FSTRIDE = 1 << 20
NC = 2
NS = 16
NW = NC * NS
BPW = BATCH // NW
CHUNK = 128
NCHUNK = BPW // CHUNK
IDX_ROWS = BATCH // CHUNK

UK = 8
UB = FSTRIDE // UK


def _untile_body(*refs):
    nt = len(refs) // 9
    ins = refs[:nt]
    outs = refs[nt:]
    for t in range(nt):
        for f in range(EMB):
            outs[t * EMB + f][...] = ins[t][f, :]


def _tc_untile(*tables):
    nt = len(tables)
    in_spec = pl.BlockSpec((EMB, UB), lambda k: (0, k))
    out_spec = pl.BlockSpec((UB,), lambda k: (k,))
    return pl.pallas_call(
        _untile_body,
        grid=(UK,),
        in_specs=[in_spec] * nt,
        out_specs=[out_spec] * (EMB * nt),
        out_shape=[jax.ShapeDtypeStruct((FSTRIDE,), jnp.float32)] * (EMB * nt),
    )(*tables)


def _sc_gather(idx2d, feats):
    mesh = plsc.VectorSubcoreMesh(core_axis_name="c", subcore_axis_name="s")
    out_t = [jax.ShapeDtypeStruct((EMB, BATCH), jnp.float32)] * 2

    @functools.partial(
        pl.kernel,
        mesh=mesh,
        out_type=out_t,
        compiler_params=pltpu.CompilerParams(use_tc_tiling_on_sc=False),
        scratch_types=[
            pltpu.VMEM((NCHUNK, CHUNK), jnp.int32),
            pltpu.VMEM((EMB, BPW), jnp.float32),
            pltpu.VMEM((EMB, BPW), jnp.float32),
            pltpu.SemaphoreType.DMA,
        ],
    )
    def k(i_hbm, *rest):
        s = rest[:16]
        o_mlp, o_mf = rest[16:18]
        idx, r_mlp, r_mf, sem = rest[18:]
        wid = lax.axis_index("s") * NC + lax.axis_index("c")
        base = wid * BPW
        row0 = wid * NCHUNK
        pltpu.sync_copy(i_hbm.at[pl.ds(row0, NCHUNK)], idx)
        copies = []
        for t, rbuf in enumerate((r_mlp, r_mf)):
            for f in range(EMB):
                src = s[t * EMB + f]
                for g in range(NCHUNK):
                    copies.append(pltpu.async_copy(
                        src.at[idx.at[g]],
                        rbuf.at[f, pl.ds(g * CHUNK, CHUNK)], sem))
        for c in copies:
            c.wait()
        osl = pl.ds(base, BPW)
        pltpu.sync_copy(r_mlp, o_mlp.at[:, osl])
        pltpu.sync_copy(r_mf, o_mf.at[:, osl])

    return k(idx2d, *feats)


BT = 2048


def _tc_body(um, im, uf, itf, w1u, w1i, b1c, w2, b2c, wah, waf, bac, out):
    h = jnp.maximum(
        jnp.dot(w1u[...], um[...], preferred_element_type=jnp.float32)
        + jnp.dot(w1i[...], im[...], preferred_element_type=jnp.float32)
        + b1c[...], 0.0)
    h2 = jnp.maximum(
        jnp.dot(w2[...], h, preferred_element_type=jnp.float32) + b2c[...], 0.0)
    mf = uf[...] * itf[...]
    logits = (jnp.dot(wah[...], h2, preferred_element_type=jnp.float32)
              + jnp.dot(waf[...], mf, preferred_element_type=jnp.float32)
              + bac[...])
    out[...] = jax.nn.sigmoid(logits)


def _tc_dense(u_mlp, i_mlp, u_mf, i_mf, w1u, w1i, b1c, w2, b2c, wah, waf, bac):
    grid = BATCH // BT
    emb_spec = pl.BlockSpec((EMB, BT), lambda i: (0, i))

    def wspec(shape):
        return pl.BlockSpec(shape, lambda i: (0, 0))

    return pl.pallas_call(
        _tc_body,
        grid=(grid,),
        in_specs=[
            emb_spec, emb_spec, emb_spec, emb_spec,
            wspec((32, EMB)), wspec((32, EMB)), wspec((32, 1)),
            wspec((EMB, 32)), wspec((EMB, 1)),
            wspec((1, EMB)), wspec((1, EMB)), wspec((1, 1)),
        ],
        out_specs=pl.BlockSpec((1, BT), lambda i: (0, i)),
        out_shape=jax.ShapeDtypeStruct((1, BATCH), jnp.float32),
    )(u_mlp, i_mlp, u_mf, i_mf, w1u, w1i, b1c, w2, b2c, wah, waf, bac)


def kernel(user_input, item_input, emb_user_mlp, emb_item_mlp,
           emb_user_mf, emb_item_mf, W1, b1, W2, b2, Wa, ba):
    u2d = user_input.astype(jnp.int32).reshape(IDX_ROWS, CHUNK)
    i2d = item_input.astype(jnp.int32).reshape(IDX_ROWS, CHUNK)
    feats_u = _tc_untile(emb_user_mlp.T, emb_user_mf.T)
    gum, guf = _sc_gather(u2d, feats_u)
    feats_i = _tc_untile(emb_item_mlp.T, emb_item_mf.T)
    gim, gif = _sc_gather(i2d, feats_i)
    out_t = _tc_dense(
        gum, gim, guf, gif,
        W1[:EMB].T, W1[EMB:].T, b1.reshape(32, 1),
        W2.T, b2.reshape(EMB, 1),
        Wa[:EMB].T, Wa[EMB:].T, ba.reshape(1, 1))
    return out_t.reshape(BATCH, 1)
